# baseline (speedup 1.0000x reference)
import functools

import jax
import jax.numpy as jnp
from jax import lax
from jax.experimental import pallas as pl
from jax.experimental.pallas import tpu as pltpu
from jax.experimental.pallas import tpu_sc as plsc

N = 10000
E = 320000
D = 128
DP = D // 2
H = 128
RB = 16

NC = 2
NS = 16
NW = NC * NS
CH = 128
NCHUNK = E // CH
ITERS = (NCHUNK + NW - 1) // NW

NSLAB = 4
ES = E // NSLAB
SCHUNK = ES // CH
SITERS = (SCHUNK + NW - 1) // NW
BLK = 2000
BPS = ES // BLK
GROWS = 80
NGROUP = N // GROWS
GITER = (NGROUP + NS - 1) // NS

_SC_MESH = dict(core_axis_name="c", subcore_axis_name="s",
                num_cores=NC, num_subcores=NS)


def _gather_body(node_hbm, src_hbm, dst_hbm, vi_hbm, vj_hbm,
                 idx_s0, idx_d0, rows_s0, rows_d0,
                 idx_s1, idx_d1, rows_s1, rows_d1,
                 sem_g, sem_w):
    cid = lax.axis_index("c")
    sid = lax.axis_index("s")
    wid = sid * NC + cid
    idx_s = (idx_s0, idx_s1)
    idx_d = (idx_d0, idx_d1)
    rows_s = (rows_s0, rows_s1)
    rows_d = (rows_d0, rows_d1)

    def body(t, carry):
        chunk = t * NW + wid
        prev2 = (t - 2) * NW + wid

        for p in range(2):
            @pl.when((t % 2 == p) & (prev2 >= 0) & (prev2 < SCHUNK))
            def _(p=p):
                pb = prev2 * CH
                pltpu.make_async_copy(
                    rows_s[p], vi_hbm.at[pl.ds(pb, CH)], sem_w).wait()
                pltpu.make_async_copy(
                    rows_d[p], vj_hbm.at[pl.ds(pb, CH)], sem_w).wait()

            @pl.when((t % 2 == p) & (chunk < SCHUNK))
            def _(p=p):
                base = chunk * CH
                pltpu.sync_copy(src_hbm.at[pl.ds(base, CH)], idx_s[p])
                pltpu.sync_copy(dst_hbm.at[pl.ds(base, CH)], idx_d[p])
                cp_s = pltpu.async_copy(node_hbm.at[idx_s[p]], rows_s[p],
                                        sem_g)
                cp_d = pltpu.async_copy(node_hbm.at[idx_d[p]], rows_d[p],
                                        sem_g)
                cp_s.wait()
                cp_d.wait()
                pltpu.async_copy(rows_s[p], vi_hbm.at[pl.ds(base, CH)], sem_w)
                pltpu.async_copy(rows_d[p], vj_hbm.at[pl.ds(base, CH)], sem_w)

        return carry

    lax.fori_loop(0, SITERS, body, 0)

    def tail(t, carry):
        chunk = t * NW + wid

        for p in range(2):
            @pl.when((t % 2 == p) & (chunk < SCHUNK))
            def _(p=p):
                base = chunk * CH
                pltpu.make_async_copy(
                    rows_s[p], vi_hbm.at[pl.ds(base, CH)], sem_w).wait()
                pltpu.make_async_copy(
                    rows_d[p], vj_hbm.at[pl.ds(base, CH)], sem_w).wait()

        return carry

    lax.fori_loop(max(SITERS - 2, 0), SITERS, tail, 0)


def _sc_gather(node_feat, src_slab, dst_slab):
    f = pl.kernel(
        _gather_body,
        out_type=(jax.ShapeDtypeStruct((ES, D), jnp.float32),
                  jax.ShapeDtypeStruct((ES, D), jnp.float32)),
        mesh=plsc.VectorSubcoreMesh(**_SC_MESH),
        scratch_types=[
            pltpu.VMEM((CH,), jnp.int32),
            pltpu.VMEM((CH,), jnp.int32),
            pltpu.VMEM((CH, D), jnp.float32),
            pltpu.VMEM((CH, D), jnp.float32),
            pltpu.VMEM((CH,), jnp.int32),
            pltpu.VMEM((CH,), jnp.int32),
            pltpu.VMEM((CH, D), jnp.float32),
            pltpu.VMEM((CH, D), jnp.float32),
            pltpu.SemaphoreType.DMA,
            pltpu.SemaphoreType.DMA,
        ],
    )
    return f(node_feat, src_slab, dst_slab)


def _scatter_body(mess_hbm, dst_hbm, node_hbm, out_hbm,
                  acc, idx0, rows0, idx1, rows1, sem_l, sem_a):
    cid = lax.axis_index("c")
    sid = lax.axis_index("s")
    wid = sid * NC + cid
    idx = (idx0, idx1)
    rows = (rows0, rows1)

    def init_body(t, carry):
        g = t * NS + sid

        @pl.when(g < NGROUP)
        def _():
            b = g * GROWS
            pltpu.sync_copy(node_hbm.at[pl.ds(b, GROWS)],
                            acc.at[pl.ds(b, GROWS)])

        return carry

    lax.fori_loop(0, GITER, init_body, 0)
    plsc.subcore_barrier()

    def body(t, carry):
        chunk = t * NW + wid
        prev2 = (t - 2) * NW + wid

        for p in range(2):
            @pl.when((t % 2 == p) & (prev2 >= 0) & (prev2 < SCHUNK))
            def _(p=p):
                pltpu.make_async_copy(rows[p], acc.at[idx[p]], sem_a).wait()

            @pl.when((t % 2 == p) & (chunk < SCHUNK))
            def _(p=p):
                base = chunk * CH
                pltpu.sync_copy(dst_hbm.at[pl.ds(base, CH)], idx[p])
                cp = pltpu.async_copy(mess_hbm.at[pl.ds(base, CH)], rows[p],
                                      sem_l)
                cp.wait()
                pltpu.async_copy(rows[p], acc.at[idx[p]], sem_a, add=True)

        return carry

    lax.fori_loop(0, SITERS, body, 0)

    def tail(t, carry):
        chunk = t * NW + wid

        for p in range(2):
            @pl.when((t % 2 == p) & (chunk < SCHUNK))
            def _(p=p):
                pltpu.make_async_copy(rows[p], acc.at[idx[p]], sem_a).wait()

        return carry

    lax.fori_loop(max(SITERS - 2, 0), SITERS, tail, 0)
    plsc.subcore_barrier()

    def dump_body(t, carry):
        g = t * NS + sid

        @pl.when(g < NGROUP)
        def _():
            b = g * GROWS
            pltpu.sync_copy(acc.at[pl.ds(b, GROWS)],
                            out_hbm.at[cid, pl.ds(b, GROWS)])

        return carry

    lax.fori_loop(0, GITER, dump_body, 0)


def _sc_scatter(mess_slab, dst_slab, node_feat):
    f = pl.kernel(
        _scatter_body,
        out_type=jax.ShapeDtypeStruct((NC, N, D), jnp.float32),
        mesh=plsc.VectorSubcoreMesh(**_SC_MESH),
        scratch_types=[
            pltpu.VMEM_SHARED((N, D), jnp.float32),
            pltpu.VMEM((CH,), jnp.int32),
            pltpu.VMEM((CH, D), jnp.float32),
            pltpu.VMEM((CH,), jnp.int32),
            pltpu.VMEM((CH, D), jnp.float32),
            pltpu.SemaphoreType.DMA,
            pltpu.SemaphoreType.DMA,
        ],
    )
    return f(mess_slab, dst_slab, node_feat)


def _silu(x):
    return x * jax.nn.sigmoid(x)


def _mlp_body(vi_ref, vj_ref, ef_ref, rbf_ref,
              w0e_ref, b0e_ref, w1em_ref, b1em_ref, w1eg_ref, b1eg_ref,
              w0n_ref, b0n_ref, w1nm_ref, b1nm_ref, w1ng_ref, b1ng_ref,
              wew_ref, wnw_ref,
              enew_ref, mess_ref):
    ef = ef_ref[...]
    ef_bf = ef.astype(jnp.bfloat16)
    rbf = rbf_ref[...]

    xn = jnp.concatenate([vi_ref[...].astype(jnp.bfloat16),
                          vj_ref[...].astype(jnp.bfloat16)],
                         axis=1)
    w0e = w0e_ref[...]
    z = jnp.dot(xn, w0e[:2 * D], preferred_element_type=jnp.float32)
    z = z + jnp.dot(ef_bf, w0e[2 * D:], preferred_element_type=jnp.float32)
    z = z + b0e_ref[...]
    h = _silu(z[:, :H]).astype(jnp.bfloat16)
    g = _silu(z[:, H:]).astype(jnp.bfloat16)
    h2 = _silu(jnp.dot(h, w1em_ref[...], preferred_element_type=jnp.float32)
               + b1em_ref[...])
    g2 = jax.nn.sigmoid(
        jnp.dot(g, w1eg_ref[...], preferred_element_type=jnp.float32)
        + b1eg_ref[...])
    rew = jnp.dot(rbf, wew_ref[...], preferred_element_type=jnp.float32)
    mij = h2 * g2 * rew
    enew = ef + mij
    enew_ref[...] = enew

    w0n = w0n_ref[...]
    z2 = jnp.dot(xn, w0n[:2 * D], preferred_element_type=jnp.float32)
    z2 = z2 + jnp.dot(enew.astype(jnp.bfloat16), w0n[2 * D:],
                      preferred_element_type=jnp.float32)
    z2 = z2 + b0n_ref[...]
    hn = _silu(z2[:, :H]).astype(jnp.bfloat16)
    gn = _silu(z2[:, H:]).astype(jnp.bfloat16)
    hn2 = _silu(jnp.dot(hn, w1nm_ref[...], preferred_element_type=jnp.float32)
                + b1nm_ref[...])
    gn2 = jax.nn.sigmoid(
        jnp.dot(gn, w1ng_ref[...], preferred_element_type=jnp.float32)
        + b1ng_ref[...])
    rnw = jnp.dot(rbf, wnw_ref[...], preferred_element_type=jnp.float32)
    mess_ref[...] = hn2 * gn2 * rnw


def _mlp_body_alias(vi_ref, vj_ref, ef_ref, rbf_ref,
                    w0e_ref, b0e_ref, w1em_ref, b1em_ref, w1eg_ref, b1eg_ref,
                    w0n_ref, b0n_ref, w1nm_ref, b1nm_ref, w1ng_ref, b1ng_ref,
                    wew_ref, wnw_ref, enew_in,
                    enew_ref, mess_ref):
    del enew_in
    _mlp_body(vi_ref, vj_ref, ef_ref, rbf_ref,
              w0e_ref, b0e_ref, w1em_ref, b1em_ref, w1eg_ref, b1eg_ref,
              w0n_ref, b0n_ref, w1nm_ref, b1nm_ref, w1ng_ref, b1ng_ref,
              wew_ref, wnw_ref, enew_ref, mess_ref)


def _tc_mlp_slab(k, vi, vj, ef, rbf_p, weights, enew_acc):
    off = k * BPS

    def sb(i, off=off):
        return (off + i, 0)

    def full(arr):
        nd = arr.ndim
        return pl.BlockSpec(arr.shape, lambda i, nd=nd: (0,) * nd)

    in_specs = [
        pl.BlockSpec((BLK, D), lambda i: (i, 0)),
        pl.BlockSpec((BLK, D), lambda i: (i, 0)),
        pl.BlockSpec((BLK, D), sb),
        pl.BlockSpec((BLK, RB), sb),
    ] + [full(w) for w in weights]
    args = [vi, vj, ef, rbf_p, *weights]
    if k == 0:
        body = _mlp_body
        aliases = {}
    else:
        body = _mlp_body_alias
        in_specs += [pl.BlockSpec(memory_space=pl.ANY)]
        args += [enew_acc]
        aliases = {18: 0}
    out_specs = [pl.BlockSpec((BLK, D), sb),
                 pl.BlockSpec((BLK, D), lambda i: (i, 0))]
    return pl.pallas_call(
        body,
        grid=(BPS,),
        in_specs=in_specs,
        out_specs=out_specs,
        out_shape=[jax.ShapeDtypeStruct((E, D), jnp.float32),
                   jax.ShapeDtypeStruct((ES, D), jnp.float32)],
        input_output_aliases=aliases,
    )(*args)


def _combine_body(p0_ref, p1_ref, p2_ref, p3_ref, nf_ref, out_ref):
    acc = p0_ref[0] + p0_ref[1]
    acc += p1_ref[0] + p1_ref[1]
    acc += p2_ref[0] + p2_ref[1]
    acc += p3_ref[0] + p3_ref[1]
    out_ref[...] = acc - (2 * NSLAB - 1) * nf_ref[...]


def _tc_combine(partials, node_feat):
    blk = 2000
    grid = (N // blk,)
    pspec = pl.BlockSpec((NC, blk, D), lambda i: (0, i, 0))
    return pl.pallas_call(
        _combine_body,
        grid=grid,
        in_specs=[pspec, pspec, pspec, pspec,
                  pl.BlockSpec((blk, D), lambda i: (i, 0))],
        out_specs=pl.BlockSpec((blk, D), lambda i: (i, 0)),
        out_shape=jax.ShapeDtypeStruct((N, D), jnp.float32),
    )(*partials, node_feat)


def kernel(edge_feat, node_feat, edge_index, rbf, graph_attr,
           W_e_m0, b_e_m0, W_e_m1, b_e_m1, W_e_g0, b_e_g0, W_e_g1, b_e_g1,
           W_n_m0, b_n_m0, W_n_m1, b_n_m1, W_n_g0, b_n_g0, W_n_g1, b_n_g1,
           W_ew, W_nw):
    src = edge_index[0].astype(jnp.int32)
    dst = edge_index[1].astype(jnp.int32)

    rbf_p = jnp.pad(rbf, ((0, 0), (0, RB - rbf.shape[1])))
    wew_p = jnp.pad(W_ew, ((0, RB - W_ew.shape[0]), (0, 0)))
    wnw_p = jnp.pad(W_nw, ((0, RB - W_nw.shape[0]), (0, 0)))

    w0e = jnp.concatenate([W_e_m0, W_e_g0], axis=1)
    b0e = jnp.concatenate([b_e_m0, b_e_g0])[None, :]
    w0n = jnp.concatenate([W_n_m0, W_n_g0], axis=1)
    b0n = jnp.concatenate([b_n_m0, b_n_g0])[None, :]
    weights = [
        w0e.astype(jnp.bfloat16), b0e,
        W_e_m1.astype(jnp.bfloat16), b_e_m1[None, :],
        W_e_g1.astype(jnp.bfloat16), b_e_g1[None, :],
        w0n.astype(jnp.bfloat16), b0n,
        W_n_m1.astype(jnp.bfloat16), b_n_m1[None, :],
        W_n_g1.astype(jnp.bfloat16), b_n_g1[None, :],
        wew_p, wnw_p,
    ]

    edge_new = None
    partials = []
    for k in range(NSLAB):
        sl = slice(k * ES, (k + 1) * ES)
        dst_k = dst[sl]
        vi_k, vj_k = _sc_gather(node_feat, src[sl], dst_k)
        edge_new, mess_k = _tc_mlp_slab(k, vi_k, vj_k, edge_feat, rbf_p,
                                        weights, edge_new)
        partials.append(_sc_scatter(mess_k, dst_k, node_feat))

    node_new = _tc_combine(partials, node_feat)
    return (edge_new, node_new, graph_attr)


# speedup vs baseline: 1.0008x; 1.0008x over previous
import functools

import jax
import jax.numpy as jnp
from jax import lax
from jax.experimental import pallas as pl
from jax.experimental.pallas import tpu as pltpu
from jax.experimental.pallas import tpu_sc as plsc

N = 10000
E = 320000
D = 128
DP = D // 2
H = 128
RB = 16

NC = 2
NS = 16
NW = NC * NS
CH = 128
NCHUNK = E // CH
ITERS = (NCHUNK + NW - 1) // NW

NSLAB = 4
ES = E // NSLAB
SCHUNK = ES // CH
SITERS = (SCHUNK + NW - 1) // NW
BLK = 2000
BPS = ES // BLK
GROWS = 80
NGROUP = N // GROWS
GITER = (NGROUP + NS - 1) // NS

_SC_MESH = dict(core_axis_name="c", subcore_axis_name="s",
                num_cores=NC, num_subcores=NS)


def _gather_body(node_hbm, src_hbm, dst_hbm, vi_hbm, vj_hbm,
                 idx_s0, idx_d0, rows_s0, rows_d0,
                 idx_s1, idx_d1, rows_s1, rows_d1,
                 sem_g, sem_w):
    cid = lax.axis_index("c")
    sid = lax.axis_index("s")
    wid = sid * NC + cid
    idx_s = (idx_s0, idx_s1)
    idx_d = (idx_d0, idx_d1)
    rows_s = (rows_s0, rows_s1)
    rows_d = (rows_d0, rows_d1)

    def body(t, carry):
        chunk = t * NW + wid
        prev2 = (t - 2) * NW + wid

        for p in range(2):
            @pl.when((t % 2 == p) & (prev2 >= 0) & (prev2 < SCHUNK))
            def _(p=p):
                pb = prev2 * CH
                pltpu.make_async_copy(
                    rows_s[p], vi_hbm.at[pl.ds(pb, CH)], sem_w).wait()
                pltpu.make_async_copy(
                    rows_d[p], vj_hbm.at[pl.ds(pb, CH)], sem_w).wait()

            @pl.when((t % 2 == p) & (chunk < SCHUNK))
            def _(p=p):
                base = chunk * CH
                pltpu.sync_copy(src_hbm.at[pl.ds(base, CH)], idx_s[p])
                pltpu.sync_copy(dst_hbm.at[pl.ds(base, CH)], idx_d[p])
                cp_s = pltpu.async_copy(node_hbm.at[idx_s[p]], rows_s[p],
                                        sem_g)
                cp_d = pltpu.async_copy(node_hbm.at[idx_d[p]], rows_d[p],
                                        sem_g)
                cp_s.wait()
                cp_d.wait()
                pltpu.async_copy(rows_s[p], vi_hbm.at[pl.ds(base, CH)], sem_w)
                pltpu.async_copy(rows_d[p], vj_hbm.at[pl.ds(base, CH)], sem_w)

        return carry

    lax.fori_loop(0, SITERS, body, 0)

    def tail(t, carry):
        chunk = t * NW + wid

        for p in range(2):
            @pl.when((t % 2 == p) & (chunk < SCHUNK))
            def _(p=p):
                base = chunk * CH
                pltpu.make_async_copy(
                    rows_s[p], vi_hbm.at[pl.ds(base, CH)], sem_w).wait()
                pltpu.make_async_copy(
                    rows_d[p], vj_hbm.at[pl.ds(base, CH)], sem_w).wait()

        return carry

    lax.fori_loop(max(SITERS - 2, 0), SITERS, tail, 0)


def _sc_gather(node_feat, src_slab, dst_slab):
    f = pl.kernel(
        _gather_body,
        out_type=(jax.ShapeDtypeStruct((ES, D), jnp.float32),
                  jax.ShapeDtypeStruct((ES, D), jnp.float32)),
        mesh=plsc.VectorSubcoreMesh(**_SC_MESH),
        scratch_types=[
            pltpu.VMEM((CH,), jnp.int32),
            pltpu.VMEM((CH,), jnp.int32),
            pltpu.VMEM((CH, D), jnp.float32),
            pltpu.VMEM((CH, D), jnp.float32),
            pltpu.VMEM((CH,), jnp.int32),
            pltpu.VMEM((CH,), jnp.int32),
            pltpu.VMEM((CH, D), jnp.float32),
            pltpu.VMEM((CH, D), jnp.float32),
            pltpu.SemaphoreType.DMA,
            pltpu.SemaphoreType.DMA,
        ],
    )
    return f(node_feat, src_slab, dst_slab)


def _scatter_body(mess_hbm, dst_hbm, node_hbm, out_hbm,
                  acc, idx0, rows0, idx1, rows1, sem_l, sem_a):
    cid = lax.axis_index("c")
    sid = lax.axis_index("s")
    wid = sid * NC + cid
    idx = (idx0, idx1)
    rows = (rows0, rows1)

    def init_body(t, carry):
        g = t * NS + sid

        @pl.when(g < NGROUP)
        def _():
            b = g * GROWS
            pltpu.sync_copy(node_hbm.at[pl.ds(b, GROWS)],
                            acc.at[pl.ds(b, GROWS)])

        return carry

    lax.fori_loop(0, GITER, init_body, 0)
    plsc.subcore_barrier()

    def body(t, carry):
        chunk = t * NW + wid
        prev2 = (t - 2) * NW + wid

        for p in range(2):
            @pl.when((t % 2 == p) & (prev2 >= 0) & (prev2 < SCHUNK))
            def _(p=p):
                pltpu.make_async_copy(rows[p], acc.at[idx[p]], sem_a).wait()

            @pl.when((t % 2 == p) & (chunk < SCHUNK))
            def _(p=p):
                base = chunk * CH
                pltpu.sync_copy(dst_hbm.at[pl.ds(base, CH)], idx[p])
                cp = pltpu.async_copy(mess_hbm.at[pl.ds(base, CH)], rows[p],
                                      sem_l)
                cp.wait()
                pltpu.async_copy(rows[p], acc.at[idx[p]], sem_a, add=True)

        return carry

    lax.fori_loop(0, SITERS, body, 0)

    def tail(t, carry):
        chunk = t * NW + wid

        for p in range(2):
            @pl.when((t % 2 == p) & (chunk < SCHUNK))
            def _(p=p):
                pltpu.make_async_copy(rows[p], acc.at[idx[p]], sem_a).wait()

        return carry

    lax.fori_loop(max(SITERS - 2, 0), SITERS, tail, 0)
    plsc.subcore_barrier()

    def dump_body(t, carry):
        g = t * NS + sid

        @pl.when(g < NGROUP)
        def _():
            b = g * GROWS
            pltpu.sync_copy(acc.at[pl.ds(b, GROWS)],
                            out_hbm.at[cid, pl.ds(b, GROWS)])

        return carry

    lax.fori_loop(0, GITER, dump_body, 0)


def _sc_scatter(mess_slab, dst_slab, node_feat):
    f = pl.kernel(
        _scatter_body,
        out_type=jax.ShapeDtypeStruct((NC, N, D), jnp.float32),
        mesh=plsc.VectorSubcoreMesh(**_SC_MESH),
        scratch_types=[
            pltpu.VMEM_SHARED((N, D), jnp.float32),
            pltpu.VMEM((CH,), jnp.int32),
            pltpu.VMEM((CH, D), jnp.float32),
            pltpu.VMEM((CH,), jnp.int32),
            pltpu.VMEM((CH, D), jnp.float32),
            pltpu.SemaphoreType.DMA,
            pltpu.SemaphoreType.DMA,
        ],
    )
    return f(mess_slab, dst_slab, node_feat)


def _silu(x):
    return x * jax.nn.sigmoid(x)


def _mlp_body(vi_ref, vj_ref, ef_ref, rbf_ref,
              w0e_ref, b0e_ref, w1em_ref, b1em_ref, w1eg_ref, b1eg_ref,
              w0n_ref, b0n_ref, w1nm_ref, b1nm_ref, w1ng_ref, b1ng_ref,
              wew_ref, wnw_ref,
              enew_ref, mess_ref):
    ef = ef_ref[...]
    ef_bf = ef.astype(jnp.bfloat16)
    rbf = rbf_ref[...]

    xn = jnp.concatenate([vi_ref[...].astype(jnp.bfloat16),
                          vj_ref[...].astype(jnp.bfloat16)],
                         axis=1)
    w0e = w0e_ref[...]
    z = jnp.dot(xn, w0e[:2 * D], preferred_element_type=jnp.float32)
    z = z + jnp.dot(ef_bf, w0e[2 * D:], preferred_element_type=jnp.float32)
    z = z + b0e_ref[...]
    h = _silu(z[:, :H]).astype(jnp.bfloat16)
    g = _silu(z[:, H:]).astype(jnp.bfloat16)
    h2 = _silu(jnp.dot(h, w1em_ref[...], preferred_element_type=jnp.float32)
               + b1em_ref[...])
    g2 = jax.nn.sigmoid(
        jnp.dot(g, w1eg_ref[...], preferred_element_type=jnp.float32)
        + b1eg_ref[...])
    rew = jnp.dot(rbf, wew_ref[...], preferred_element_type=jnp.float32)
    mij = h2 * g2 * rew
    enew = ef + mij
    enew_ref[...] = enew

    w0n = w0n_ref[...]
    z2 = jnp.dot(xn, w0n[:2 * D], preferred_element_type=jnp.float32)
    z2 = z2 + jnp.dot(enew.astype(jnp.bfloat16), w0n[2 * D:],
                      preferred_element_type=jnp.float32)
    z2 = z2 + b0n_ref[...]
    hn = _silu(z2[:, :H]).astype(jnp.bfloat16)
    gn = _silu(z2[:, H:]).astype(jnp.bfloat16)
    hn2 = _silu(jnp.dot(hn, w1nm_ref[...], preferred_element_type=jnp.float32)
                + b1nm_ref[...])
    gn2 = jax.nn.sigmoid(
        jnp.dot(gn, w1ng_ref[...], preferred_element_type=jnp.float32)
        + b1ng_ref[...])
    rnw = jnp.dot(rbf, wnw_ref[...], preferred_element_type=jnp.float32)
    mess_ref[...] = hn2 * gn2 * rnw


def _mlp_body_alias(vi_ref, vj_ref, ef_ref, rbf_ref,
                    w0e_ref, b0e_ref, w1em_ref, b1em_ref, w1eg_ref, b1eg_ref,
                    w0n_ref, b0n_ref, w1nm_ref, b1nm_ref, w1ng_ref, b1ng_ref,
                    wew_ref, wnw_ref, enew_in,
                    enew_ref, mess_ref):
    del enew_in
    _mlp_body(vi_ref, vj_ref, ef_ref, rbf_ref,
              w0e_ref, b0e_ref, w1em_ref, b1em_ref, w1eg_ref, b1eg_ref,
              w0n_ref, b0n_ref, w1nm_ref, b1nm_ref, w1ng_ref, b1ng_ref,
              wew_ref, wnw_ref, enew_ref, mess_ref)


def _tc_mlp_slab(k, vi, vj, ef, rbf_p, weights, enew_acc):
    off = k * BPS

    def sb(i, off=off):
        return (off + i, 0)

    def full(arr):
        nd = arr.ndim
        return pl.BlockSpec(arr.shape, lambda i, nd=nd: (0,) * nd)

    in_specs = [
        pl.BlockSpec((BLK, D), lambda i: (i, 0)),
        pl.BlockSpec((BLK, D), lambda i: (i, 0)),
        pl.BlockSpec((BLK, D), sb),
        pl.BlockSpec((BLK, RB), sb),
    ] + [full(w) for w in weights]
    args = [vi, vj, ef, rbf_p, *weights]
    if k == 0:
        body = _mlp_body
        aliases = {}
    else:
        body = _mlp_body_alias
        in_specs += [pl.BlockSpec(memory_space=pl.ANY)]
        args += [enew_acc]
        aliases = {18: 0}
    out_specs = [pl.BlockSpec((BLK, D), sb),
                 pl.BlockSpec((BLK, D), lambda i: (i, 0))]
    return pl.pallas_call(
        body,
        grid=(BPS,),
        in_specs=in_specs,
        out_specs=out_specs,
        out_shape=[jax.ShapeDtypeStruct((E, D), jnp.float32),
                   jax.ShapeDtypeStruct((ES, D), jnp.float32)],
        input_output_aliases=aliases,
    )(*args)


def _combine_body(p0_ref, p1_ref, p2_ref, p3_ref, nf_ref, out_ref):
    acc = p0_ref[0] + p0_ref[1]
    acc += p1_ref[0] + p1_ref[1]
    acc += p2_ref[0] + p2_ref[1]
    acc += p3_ref[0] + p3_ref[1]
    out_ref[...] = acc - (2 * NSLAB - 1) * nf_ref[...]


def _tc_combine(partials, node_feat):
    blk = 2000
    grid = (N // blk,)
    pspec = pl.BlockSpec((NC, blk, D), lambda i: (0, i, 0))
    return pl.pallas_call(
        _combine_body,
        grid=grid,
        in_specs=[pspec, pspec, pspec, pspec,
                  pl.BlockSpec((blk, D), lambda i: (i, 0))],
        out_specs=pl.BlockSpec((blk, D), lambda i: (i, 0)),
        out_shape=jax.ShapeDtypeStruct((N, D), jnp.float32),
    )(*partials, node_feat)


def kernel(edge_feat, node_feat, edge_index, rbf, graph_attr,
           W_e_m0, b_e_m0, W_e_m1, b_e_m1, W_e_g0, b_e_g0, W_e_g1, b_e_g1,
           W_n_m0, b_n_m0, W_n_m1, b_n_m1, W_n_g0, b_n_g0, W_n_g1, b_n_g1,
           W_ew, W_nw):
    src = edge_index[0].astype(jnp.int32)
    dst = edge_index[1].astype(jnp.int32)

    rbf_p = jnp.pad(rbf, ((0, 0), (0, RB - rbf.shape[1])))
    wew_p = jnp.pad(W_ew, ((0, RB - W_ew.shape[0]), (0, 0)))
    wnw_p = jnp.pad(W_nw, ((0, RB - W_nw.shape[0]), (0, 0)))

    w0e = jnp.concatenate([W_e_m0, W_e_g0], axis=1)
    b0e = jnp.concatenate([b_e_m0, b_e_g0])[None, :]
    w0n = jnp.concatenate([W_n_m0, W_n_g0], axis=1)
    b0n = jnp.concatenate([b_n_m0, b_n_g0])[None, :]
    weights = [
        w0e.astype(jnp.bfloat16), b0e,
        W_e_m1.astype(jnp.bfloat16), b_e_m1[None, :],
        W_e_g1.astype(jnp.bfloat16), b_e_g1[None, :],
        w0n.astype(jnp.bfloat16), b0n,
        W_n_m1.astype(jnp.bfloat16), b_n_m1[None, :],
        W_n_g1.astype(jnp.bfloat16), b_n_g1[None, :],
        wew_p, wnw_p,
    ]

    def _slab(k, a):
        sl = slice(k * ES, (k + 1) * ES)
        return a[sl]

    gathered = {}
    for k in range(min(2, NSLAB)):
        gathered[k] = _sc_gather(node_feat, _slab(k, src), _slab(k, dst))

    edge_new = None
    partials = []
    for k in range(NSLAB):
        vi_k, vj_k = gathered.pop(k)
        edge_new, mess_k = _tc_mlp_slab(k, vi_k, vj_k, edge_feat, rbf_p,
                                        weights, edge_new)
        if k + 2 < NSLAB:
            gathered[k + 2] = _sc_gather(node_feat, _slab(k + 2, src),
                                         _slab(k + 2, dst))
        partials.append(_sc_scatter(mess_k, _slab(k, dst), node_feat))

    node_new = _tc_combine(partials, node_feat)
    return (edge_new, node_new, graph_attr)


# speedup vs baseline: 1.0474x; 1.0466x over previous
import functools

import jax
import jax.numpy as jnp
from jax import lax
from jax.experimental import pallas as pl
from jax.experimental.pallas import tpu as pltpu
from jax.experimental.pallas import tpu_sc as plsc

N = 10000
E = 320000
D = 128
H = 128
DEG = 9

NC = 2
NS = 16
NW = NC * NS
CH = 128

NSLAB = 4
ES = E // NSLAB
SCHUNK = ES // CH
SITERS = (SCHUNK + NW - 1) // NW
BLK = 2000
BPS = ES // BLK
GROWS = 80
NGROUP = N // GROWS
GITER = (NGROUP + NS - 1) // NS

_SC_MESH = dict(core_axis_name="c", subcore_axis_name="s",
                num_cores=NC, num_subcores=NS)


def _sc_step_body(has_g, has_s, goff, soff, ch, *refs):
    schunk = ES // ch
    siters = (schunk + NW - 1) // NW
    it = iter(refs)
    node_hbm = next(it)
    src_hbm = next(it)
    dst_hbm = next(it)
    mess_hbm = next(it) if has_s else None
    if has_g:
        vi_hbm = next(it)
        vj_hbm = next(it)
    if has_s:
        part_hbm = next(it)
    if has_g:
        idx_s = (next(it), next(it))
        idx_d = (next(it), next(it))
        rows_s = (next(it), next(it))
        rows_d = (next(it), next(it))
        sem_g = next(it)
        sem_w = next(it)
    if has_s:
        acc = next(it)
        idx_m = (next(it), next(it))
        rows_m = (next(it), next(it))
        sem_l = next(it)
        sem_a = next(it)

    cid = lax.axis_index("c")
    sid = lax.axis_index("s")
    wid = sid * NC + cid

    if has_s:
        def init_body(t, carry):
            g = t * NS + sid

            @pl.when(g < NGROUP)
            def _():
                b = g * GROWS
                pltpu.sync_copy(node_hbm.at[pl.ds(b, GROWS)],
                                acc.at[pl.ds(b, GROWS)])

            return carry

        lax.fori_loop(0, GITER, init_body, 0)
        plsc.subcore_barrier()

    def body(t, carry):
        lc = t * NW + wid
        lp2 = (t - 2) * NW + wid

        for p in range(2):
            if has_g:
                @pl.when((t % 2 == p) & (lp2 >= 0) & (lp2 < schunk))
                def _(p=p):
                    pb = lp2 * ch
                    pltpu.make_async_copy(
                        rows_s[p], vi_hbm.at[pl.ds(pb, ch)], sem_w).wait()
                    pltpu.make_async_copy(
                        rows_d[p], vj_hbm.at[pl.ds(pb, ch)], sem_w).wait()

                @pl.when((t % 2 == p) & (lc < schunk))
                def _(p=p):
                    gbase = (goff + lc) * ch
                    lbase = lc * ch
                    pltpu.sync_copy(src_hbm.at[pl.ds(gbase, ch)], idx_s[p])
                    pltpu.sync_copy(dst_hbm.at[pl.ds(gbase, ch)], idx_d[p])
                    cp_s = pltpu.async_copy(node_hbm.at[idx_s[p]], rows_s[p],
                                            sem_g)
                    cp_d = pltpu.async_copy(node_hbm.at[idx_d[p]], rows_d[p],
                                            sem_g)
                    cp_s.wait()
                    cp_d.wait()
                    pltpu.async_copy(rows_s[p], vi_hbm.at[pl.ds(lbase, ch)],
                                     sem_w)
                    pltpu.async_copy(rows_d[p], vj_hbm.at[pl.ds(lbase, ch)],
                                     sem_w)

            if has_s:
                @pl.when((t % 2 == p) & (lp2 >= 0) & (lp2 < schunk))
                def _(p=p):
                    pltpu.make_async_copy(rows_m[p], acc.at[idx_m[p]],
                                          sem_a).wait()

                @pl.when((t % 2 == p) & (lc < schunk))
                def _(p=p):
                    sbase = (soff + lc) * ch
                    lbase = lc * ch
                    pltpu.sync_copy(dst_hbm.at[pl.ds(sbase, ch)], idx_m[p])
                    cp = pltpu.async_copy(mess_hbm.at[pl.ds(lbase, ch)],
                                          rows_m[p], sem_l)
                    cp.wait()
                    pltpu.async_copy(rows_m[p], acc.at[idx_m[p]], sem_a,
                                     add=True)

        return carry

    lax.fori_loop(0, siters, body, 0)

    def tail(t, carry):
        lc = t * NW + wid

        for p in range(2):
            if has_g:
                @pl.when((t % 2 == p) & (lc < schunk))
                def _(p=p):
                    lbase = lc * ch
                    pltpu.make_async_copy(
                        rows_s[p], vi_hbm.at[pl.ds(lbase, ch)], sem_w).wait()
                    pltpu.make_async_copy(
                        rows_d[p], vj_hbm.at[pl.ds(lbase, ch)], sem_w).wait()

            if has_s:
                @pl.when((t % 2 == p) & (lc < schunk))
                def _(p=p):
                    pltpu.make_async_copy(rows_m[p], acc.at[idx_m[p]],
                                          sem_a).wait()

        return carry

    lax.fori_loop(max(siters - 2, 0), siters, tail, 0)

    if has_s:
        plsc.subcore_barrier()

        def dump_body(t, carry):
            g = t * NS + sid

            @pl.when(g < NGROUP)
            def _():
                b = g * GROWS
                pltpu.sync_copy(acc.at[pl.ds(b, GROWS)],
                                part_hbm.at[cid, pl.ds(b, GROWS)])

            return carry

        lax.fori_loop(0, GITER, dump_body, 0)


def _sc_step(gather_slab, scatter_slab, node_feat, src, dst, mess):
    has_g = gather_slab is not None
    has_s = scatter_slab is not None
    ch = 64 if (has_g and has_s) else CH
    schunk = ES // ch
    goff = (gather_slab or 0) * schunk
    soff = (scatter_slab or 0) * schunk

    out_type = []
    if has_g:
        out_type += [jax.ShapeDtypeStruct((ES, D), jnp.float32),
                     jax.ShapeDtypeStruct((ES, D), jnp.float32)]
    if has_s:
        out_type += [jax.ShapeDtypeStruct((NC, N, D), jnp.float32)]

    scratch = []
    if has_g:
        scratch += [pltpu.VMEM((ch,), jnp.int32)] * 4
        scratch += [pltpu.VMEM((ch, D), jnp.float32)] * 4
        scratch += [pltpu.SemaphoreType.DMA] * 2
    if has_s:
        scratch += [pltpu.VMEM_SHARED((N, D), jnp.float32)]
        scratch += [pltpu.VMEM((ch,), jnp.int32)] * 2
        scratch += [pltpu.VMEM((ch, D), jnp.float32)] * 2
        scratch += [pltpu.SemaphoreType.DMA] * 2

    body = functools.partial(_sc_step_body, has_g, has_s, goff, soff, ch)
    f = pl.kernel(
        body,
        out_type=tuple(out_type),
        mesh=plsc.VectorSubcoreMesh(**_SC_MESH),
        scratch_types=scratch,
    )
    args = [node_feat, src, dst]
    if has_s:
        args.append(mess)
    res = f(*args)
    if has_s and not has_g and isinstance(res, (tuple, list)):
        return res[0]
    return res


def _silu(x):
    return x * jax.nn.sigmoid(x)


def _mlp_body(vi_ref, vj_ref, ef_ref, rbf_ref,
              w0e_ref, b0e_ref, w1em_ref, b1em_ref, w1eg_ref, b1eg_ref,
              w0n_ref, b0n_ref, w1nm_ref, b1nm_ref, w1ng_ref, b1ng_ref,
              wew_ref, wnw_ref,
              enew_ref, mess_ref):
    ef = ef_ref[...]
    ef_bf = ef.astype(jnp.bfloat16)
    rbf = rbf_ref[...]

    xn = jnp.concatenate([vi_ref[...].astype(jnp.bfloat16),
                          vj_ref[...].astype(jnp.bfloat16)],
                         axis=1)
    w0e = w0e_ref[...]
    z = jnp.dot(xn, w0e[:2 * D], preferred_element_type=jnp.float32)
    z = z + jnp.dot(ef_bf, w0e[2 * D:], preferred_element_type=jnp.float32)
    z = z + b0e_ref[...]
    h = _silu(z[:, :H]).astype(jnp.bfloat16)
    g = _silu(z[:, H:]).astype(jnp.bfloat16)
    h2 = _silu(jnp.dot(h, w1em_ref[...], preferred_element_type=jnp.float32)
               + b1em_ref[...])
    g2 = jax.nn.sigmoid(
        jnp.dot(g, w1eg_ref[...], preferred_element_type=jnp.float32)
        + b1eg_ref[...])
    rew = jnp.dot(rbf, wew_ref[...], preferred_element_type=jnp.float32)
    mij = h2 * g2 * rew
    enew = ef + mij
    enew_ref[...] = enew

    w0n = w0n_ref[...]
    z2 = jnp.dot(xn, w0n[:2 * D], preferred_element_type=jnp.float32)
    z2 = z2 + jnp.dot(enew.astype(jnp.bfloat16), w0n[2 * D:],
                      preferred_element_type=jnp.float32)
    z2 = z2 + b0n_ref[...]
    hn = _silu(z2[:, :H]).astype(jnp.bfloat16)
    gn = _silu(z2[:, H:]).astype(jnp.bfloat16)
    hn2 = _silu(jnp.dot(hn, w1nm_ref[...], preferred_element_type=jnp.float32)
                + b1nm_ref[...])
    gn2 = jax.nn.sigmoid(
        jnp.dot(gn, w1ng_ref[...], preferred_element_type=jnp.float32)
        + b1ng_ref[...])
    rnw = jnp.dot(rbf, wnw_ref[...], preferred_element_type=jnp.float32)
    mess_ref[...] = hn2 * gn2 * rnw


def _mlp_body_alias(vi_ref, vj_ref, ef_ref, rbf_ref,
                    w0e_ref, b0e_ref, w1em_ref, b1em_ref, w1eg_ref, b1eg_ref,
                    w0n_ref, b0n_ref, w1nm_ref, b1nm_ref, w1ng_ref, b1ng_ref,
                    wew_ref, wnw_ref, enew_in,
                    enew_ref, mess_ref):
    del enew_in
    _mlp_body(vi_ref, vj_ref, ef_ref, rbf_ref,
              w0e_ref, b0e_ref, w1em_ref, b1em_ref, w1eg_ref, b1eg_ref,
              w0n_ref, b0n_ref, w1nm_ref, b1nm_ref, w1ng_ref, b1ng_ref,
              wew_ref, wnw_ref, enew_ref, mess_ref)


def _tc_mlp_slab(k, vi, vj, ef, rbf, weights, enew_acc):
    off = k * BPS

    def sb(i, off=off):
        return (off + i, 0)

    def full(arr):
        nd = arr.ndim
        return pl.BlockSpec(arr.shape, lambda i, nd=nd: (0,) * nd)

    in_specs = [
        pl.BlockSpec((BLK, D), lambda i: (i, 0)),
        pl.BlockSpec((BLK, D), lambda i: (i, 0)),
        pl.BlockSpec((BLK, D), sb),
        pl.BlockSpec((BLK, DEG), sb),
    ] + [full(w) for w in weights]
    args = [vi, vj, ef, rbf, *weights]
    if k == 0:
        body = _mlp_body
        aliases = {}
    else:
        body = _mlp_body_alias
        in_specs += [pl.BlockSpec(memory_space=pl.ANY)]
        args += [enew_acc]
        aliases = {18: 0}
    out_specs = [pl.BlockSpec((BLK, D), sb),
                 pl.BlockSpec((BLK, D), lambda i: (i, 0))]
    return pl.pallas_call(
        body,
        grid=(BPS,),
        in_specs=in_specs,
        out_specs=out_specs,
        out_shape=[jax.ShapeDtypeStruct((E, D), jnp.float32),
                   jax.ShapeDtypeStruct((ES, D), jnp.float32)],
        input_output_aliases=aliases,
    )(*args)


def _combine_body(p0_ref, p1_ref, p2_ref, p3_ref, nf_ref, out_ref):
    acc = p0_ref[0] + p0_ref[1]
    acc += p1_ref[0] + p1_ref[1]
    acc += p2_ref[0] + p2_ref[1]
    acc += p3_ref[0] + p3_ref[1]
    out_ref[...] = acc - (2 * NSLAB - 1) * nf_ref[...]


def _tc_combine(partials, node_feat):
    blk = 2000
    grid = (N // blk,)
    pspec = pl.BlockSpec((NC, blk, D), lambda i: (0, i, 0))
    return pl.pallas_call(
        _combine_body,
        grid=grid,
        in_specs=[pspec, pspec, pspec, pspec,
                  pl.BlockSpec((blk, D), lambda i: (i, 0))],
        out_specs=pl.BlockSpec((blk, D), lambda i: (i, 0)),
        out_shape=jax.ShapeDtypeStruct((N, D), jnp.float32),
    )(*partials, node_feat)


def kernel(edge_feat, node_feat, edge_index, rbf, graph_attr,
           W_e_m0, b_e_m0, W_e_m1, b_e_m1, W_e_g0, b_e_g0, W_e_g1, b_e_g1,
           W_n_m0, b_n_m0, W_n_m1, b_n_m1, W_n_g0, b_n_g0, W_n_g1, b_n_g1,
           W_ew, W_nw):
    src = edge_index[0].astype(jnp.int32)
    dst = edge_index[1].astype(jnp.int32)

    w0e = jnp.concatenate([W_e_m0, W_e_g0], axis=1)
    b0e = jnp.concatenate([b_e_m0, b_e_g0])[None, :]
    w0n = jnp.concatenate([W_n_m0, W_n_g0], axis=1)
    b0n = jnp.concatenate([b_n_m0, b_n_g0])[None, :]
    weights = [
        w0e.astype(jnp.bfloat16), b0e,
        W_e_m1.astype(jnp.bfloat16), b_e_m1[None, :],
        W_e_g1.astype(jnp.bfloat16), b_e_g1[None, :],
        w0n.astype(jnp.bfloat16), b0n,
        W_n_m1.astype(jnp.bfloat16), b_n_m1[None, :],
        W_n_g1.astype(jnp.bfloat16), b_n_g1[None, :],
        W_ew, W_nw,
    ]

    gathered = {}
    for k in range(min(2, NSLAB)):
        gathered[k] = _sc_step(k, None, node_feat, src, dst, None)

    edge_new = None
    partials = []
    for k in range(NSLAB):
        vi_k, vj_k = gathered.pop(k)
        edge_new, mess_k = _tc_mlp_slab(k, vi_k, vj_k, edge_feat, rbf,
                                        weights, edge_new)
        if k + 2 < NSLAB:
            out = _sc_step(k + 2, k, node_feat, src, dst, mess_k)
            gathered[k + 2] = (out[0], out[1])
            partials.append(out[2])
        else:
            partials.append(_sc_step(None, k, node_feat, src, dst, mess_k))

    node_new = _tc_combine(partials, node_feat)
    return (edge_new, node_new, graph_attr)


# speedup vs baseline: 1.0479x; 1.0005x over previous
import functools

import jax
import jax.numpy as jnp
from jax import lax
from jax.experimental import pallas as pl
from jax.experimental.pallas import tpu as pltpu
from jax.experimental.pallas import tpu_sc as plsc

N = 10000
E = 320000
D = 128
H = 128
DEG = 9

NC = 2
NS = 16
NW = NC * NS
CH = 128

NSLAB = 4
ES = E // NSLAB
SCHUNK = ES // CH
SITERS = (SCHUNK + NW - 1) // NW
BLK = 2000
BPS = ES // BLK
GROWS = 80
NGROUP = N // GROWS
GITER = (NGROUP + NS - 1) // NS

_SC_MESH = dict(core_axis_name="c", subcore_axis_name="s",
                num_cores=NC, num_subcores=NS)


def _sc_step_body(has_g, has_s, goff, soff, ch, sch, nslot_s, *refs):
    schunk = ES // ch
    siters = (schunk + NW - 1) // NW
    schunk_s = ES // sch
    siters_s = (schunk_s + NW - 1) // NW
    it = iter(refs)
    node_hbm = next(it)
    src_hbm = next(it)
    dst_hbm = next(it)
    mess_hbm = next(it) if has_s else None
    if has_g:
        vi_hbm = next(it)
        vj_hbm = next(it)
    if has_s:
        part_hbm = next(it)
    if has_g:
        idx_s = (next(it), next(it))
        idx_d = (next(it), next(it))
        rows_s = (next(it), next(it))
        rows_d = (next(it), next(it))
        sem_g = next(it)
        sem_w = next(it)
    if has_s:
        acc = next(it)
        idx_m = tuple(next(it) for _ in range(nslot_s))
        rows_m = tuple(next(it) for _ in range(nslot_s))
        sem_l = next(it)
        sem_a = next(it)

    cid = lax.axis_index("c")
    sid = lax.axis_index("s")
    wid = sid * NC + cid

    if has_s:
        def init_body(t, carry):
            g = t * NS + sid

            @pl.when(g < NGROUP)
            def _():
                b = g * GROWS
                pltpu.sync_copy(node_hbm.at[pl.ds(b, GROWS)],
                                acc.at[pl.ds(b, GROWS)])

            return carry

        lax.fori_loop(0, GITER, init_body, 0)
        plsc.subcore_barrier()

    def body(t, carry):
        lc = t * NW + wid
        lp2 = (t - 2) * NW + wid

        for p in range(2):
            if has_g:
                @pl.when((t % 2 == p) & (lp2 >= 0) & (lp2 < schunk))
                def _(p=p):
                    pb = lp2 * ch
                    pltpu.make_async_copy(
                        rows_s[p], vi_hbm.at[pl.ds(pb, ch)], sem_w).wait()
                    pltpu.make_async_copy(
                        rows_d[p], vj_hbm.at[pl.ds(pb, ch)], sem_w).wait()

                @pl.when((t % 2 == p) & (lc < schunk))
                def _(p=p):
                    gbase = (goff + lc) * ch
                    lbase = lc * ch
                    pltpu.sync_copy(src_hbm.at[pl.ds(gbase, ch)], idx_s[p])
                    pltpu.sync_copy(dst_hbm.at[pl.ds(gbase, ch)], idx_d[p])
                    cp_s = pltpu.async_copy(node_hbm.at[idx_s[p]], rows_s[p],
                                            sem_g)
                    cp_d = pltpu.async_copy(node_hbm.at[idx_d[p]], rows_d[p],
                                            sem_g)
                    cp_s.wait()
                    cp_d.wait()
                    pltpu.async_copy(rows_s[p], vi_hbm.at[pl.ds(lbase, ch)],
                                     sem_w)
                    pltpu.async_copy(rows_d[p], vj_hbm.at[pl.ds(lbase, ch)],
                                     sem_w)

        if has_s:
            lcs = t * NW + wid
            lps = (t - nslot_s) * NW + wid
            for p in range(nslot_s):
                @pl.when((t % nslot_s == p) & (lps >= 0) & (lps < schunk_s))
                def _(p=p):
                    pltpu.make_async_copy(rows_m[p], acc.at[idx_m[p]],
                                          sem_a).wait()

                @pl.when((t % nslot_s == p) & (lcs < schunk_s))
                def _(p=p):
                    sbase = (soff + lcs) * sch
                    lbase = lcs * sch
                    pltpu.sync_copy(dst_hbm.at[pl.ds(sbase, sch)], idx_m[p])
                    cp = pltpu.async_copy(mess_hbm.at[pl.ds(lbase, sch)],
                                          rows_m[p], sem_l)
                    cp.wait()
                    pltpu.async_copy(rows_m[p], acc.at[idx_m[p]], sem_a,
                                     add=True)

        return carry

    lax.fori_loop(0, max(siters if has_g else 0,
                         siters_s if has_s else 0), body, 0)

    if has_g:
        def tail_g(t, carry):
            lc = t * NW + wid

            for p in range(2):
                @pl.when((t % 2 == p) & (lc < schunk))
                def _(p=p):
                    lbase = lc * ch
                    pltpu.make_async_copy(
                        rows_s[p], vi_hbm.at[pl.ds(lbase, ch)], sem_w).wait()
                    pltpu.make_async_copy(
                        rows_d[p], vj_hbm.at[pl.ds(lbase, ch)], sem_w).wait()

            return carry

        lax.fori_loop(max(siters - 2, 0), siters, tail_g, 0)

    if has_s:
        def tail_s(t, carry):
            lcs = t * NW + wid

            for p in range(nslot_s):
                @pl.when((t % nslot_s == p) & (lcs < schunk_s))
                def _(p=p):
                    pltpu.make_async_copy(rows_m[p], acc.at[idx_m[p]],
                                          sem_a).wait()

            return carry

        lax.fori_loop(max(siters_s - nslot_s, 0), siters_s, tail_s, 0)

    if has_s:
        plsc.subcore_barrier()

        def dump_body(t, carry):
            g = t * NS + sid

            @pl.when(g < NGROUP)
            def _():
                b = g * GROWS
                pltpu.sync_copy(acc.at[pl.ds(b, GROWS)],
                                part_hbm.at[cid, pl.ds(b, GROWS)])

            return carry

        lax.fori_loop(0, GITER, dump_body, 0)


def _sc_step(gather_slab, scatter_slab, node_feat, src, dst, mess):
    has_g = gather_slab is not None
    has_s = scatter_slab is not None
    fused = has_g and has_s
    ch = 64 if fused else CH
    sch = 64 if fused else CH
    nslot_s = 2
    goff = (gather_slab or 0) * (ES // ch)
    soff = (scatter_slab or 0) * (ES // sch)

    out_type = []
    if has_g:
        out_type += [jax.ShapeDtypeStruct((ES, D), jnp.float32),
                     jax.ShapeDtypeStruct((ES, D), jnp.float32)]
    if has_s:
        out_type += [jax.ShapeDtypeStruct((NC, N, D), jnp.float32)]

    scratch = []
    if has_g:
        scratch += [pltpu.VMEM((ch,), jnp.int32)] * 4
        scratch += [pltpu.VMEM((ch, D), jnp.float32)] * 4
        scratch += [pltpu.SemaphoreType.DMA] * 2
    if has_s:
        scratch += [pltpu.VMEM_SHARED((N, D), jnp.float32)]
        scratch += [pltpu.VMEM((sch,), jnp.int32)] * nslot_s
        scratch += [pltpu.VMEM((sch, D), jnp.float32)] * nslot_s
        scratch += [pltpu.SemaphoreType.DMA] * 2

    body = functools.partial(_sc_step_body, has_g, has_s, goff, soff, ch,
                             sch, nslot_s)
    f = pl.kernel(
        body,
        out_type=tuple(out_type),
        mesh=plsc.VectorSubcoreMesh(**_SC_MESH),
        scratch_types=scratch,
    )
    args = [node_feat, src, dst]
    if has_s:
        args.append(mess)
    res = f(*args)
    if has_s and not has_g and isinstance(res, (tuple, list)):
        return res[0]
    return res


def _silu(x):
    return x * jax.nn.sigmoid(x)


def _mlp_body(vi_ref, vj_ref, ef_ref, rbf_ref,
              w0e_ref, b0e_ref, w1em_ref, b1em_ref, w1eg_ref, b1eg_ref,
              w0n_ref, b0n_ref, w1nm_ref, b1nm_ref, w1ng_ref, b1ng_ref,
              wew_ref, wnw_ref,
              enew_ref, mess_ref):
    ef = ef_ref[...]
    ef_bf = ef.astype(jnp.bfloat16)
    rbf = rbf_ref[...]

    xn = jnp.concatenate([vi_ref[...].astype(jnp.bfloat16),
                          vj_ref[...].astype(jnp.bfloat16)],
                         axis=1)
    w0e = w0e_ref[...]
    z = jnp.dot(xn, w0e[:2 * D], preferred_element_type=jnp.float32)
    z = z + jnp.dot(ef_bf, w0e[2 * D:], preferred_element_type=jnp.float32)
    z = z + b0e_ref[...]
    h = _silu(z[:, :H]).astype(jnp.bfloat16)
    g = _silu(z[:, H:]).astype(jnp.bfloat16)
    h2 = _silu(jnp.dot(h, w1em_ref[...], preferred_element_type=jnp.float32)
               + b1em_ref[...])
    g2 = jax.nn.sigmoid(
        jnp.dot(g, w1eg_ref[...], preferred_element_type=jnp.float32)
        + b1eg_ref[...])
    rew = jnp.dot(rbf, wew_ref[...], preferred_element_type=jnp.float32)
    mij = h2 * g2 * rew
    enew = ef + mij
    enew_ref[...] = enew

    w0n = w0n_ref[...]
    z2 = jnp.dot(xn, w0n[:2 * D], preferred_element_type=jnp.float32)
    z2 = z2 + jnp.dot(enew.astype(jnp.bfloat16), w0n[2 * D:],
                      preferred_element_type=jnp.float32)
    z2 = z2 + b0n_ref[...]
    hn = _silu(z2[:, :H]).astype(jnp.bfloat16)
    gn = _silu(z2[:, H:]).astype(jnp.bfloat16)
    hn2 = _silu(jnp.dot(hn, w1nm_ref[...], preferred_element_type=jnp.float32)
                + b1nm_ref[...])
    gn2 = jax.nn.sigmoid(
        jnp.dot(gn, w1ng_ref[...], preferred_element_type=jnp.float32)
        + b1ng_ref[...])
    rnw = jnp.dot(rbf, wnw_ref[...], preferred_element_type=jnp.float32)
    mess_ref[...] = hn2 * gn2 * rnw


def _mlp_body_alias(vi_ref, vj_ref, ef_ref, rbf_ref,
                    w0e_ref, b0e_ref, w1em_ref, b1em_ref, w1eg_ref, b1eg_ref,
                    w0n_ref, b0n_ref, w1nm_ref, b1nm_ref, w1ng_ref, b1ng_ref,
                    wew_ref, wnw_ref, enew_in,
                    enew_ref, mess_ref):
    del enew_in
    _mlp_body(vi_ref, vj_ref, ef_ref, rbf_ref,
              w0e_ref, b0e_ref, w1em_ref, b1em_ref, w1eg_ref, b1eg_ref,
              w0n_ref, b0n_ref, w1nm_ref, b1nm_ref, w1ng_ref, b1ng_ref,
              wew_ref, wnw_ref, enew_ref, mess_ref)


def _tc_mlp_slab(k, vi, vj, ef, rbf, weights, enew_acc):
    off = k * BPS

    def sb(i, off=off):
        return (off + i, 0)

    def full(arr):
        nd = arr.ndim
        return pl.BlockSpec(arr.shape, lambda i, nd=nd: (0,) * nd)

    in_specs = [
        pl.BlockSpec((BLK, D), lambda i: (i, 0)),
        pl.BlockSpec((BLK, D), lambda i: (i, 0)),
        pl.BlockSpec((BLK, D), sb),
        pl.BlockSpec((BLK, DEG), sb),
    ] + [full(w) for w in weights]
    args = [vi, vj, ef, rbf, *weights]
    if k == 0:
        body = _mlp_body
        aliases = {}
    else:
        body = _mlp_body_alias
        in_specs += [pl.BlockSpec(memory_space=pl.ANY)]
        args += [enew_acc]
        aliases = {18: 0}
    out_specs = [pl.BlockSpec((BLK, D), sb),
                 pl.BlockSpec((BLK, D), lambda i: (i, 0))]
    return pl.pallas_call(
        body,
        grid=(BPS,),
        in_specs=in_specs,
        out_specs=out_specs,
        out_shape=[jax.ShapeDtypeStruct((E, D), jnp.float32),
                   jax.ShapeDtypeStruct((ES, D), jnp.float32)],
        input_output_aliases=aliases,
    )(*args)


def _combine_body(p0_ref, p1_ref, p2_ref, p3_ref, nf_ref, out_ref):
    acc = p0_ref[0] + p0_ref[1]
    acc += p1_ref[0] + p1_ref[1]
    acc += p2_ref[0] + p2_ref[1]
    acc += p3_ref[0] + p3_ref[1]
    out_ref[...] = acc - (2 * NSLAB - 1) * nf_ref[...]


def _tc_combine(partials, node_feat):
    blk = 2000
    grid = (N // blk,)
    pspec = pl.BlockSpec((NC, blk, D), lambda i: (0, i, 0))
    return pl.pallas_call(
        _combine_body,
        grid=grid,
        in_specs=[pspec, pspec, pspec, pspec,
                  pl.BlockSpec((blk, D), lambda i: (i, 0))],
        out_specs=pl.BlockSpec((blk, D), lambda i: (i, 0)),
        out_shape=jax.ShapeDtypeStruct((N, D), jnp.float32),
    )(*partials, node_feat)


def kernel(edge_feat, node_feat, edge_index, rbf, graph_attr,
           W_e_m0, b_e_m0, W_e_m1, b_e_m1, W_e_g0, b_e_g0, W_e_g1, b_e_g1,
           W_n_m0, b_n_m0, W_n_m1, b_n_m1, W_n_g0, b_n_g0, W_n_g1, b_n_g1,
           W_ew, W_nw):
    src = edge_index[0].astype(jnp.int32)
    dst = edge_index[1].astype(jnp.int32)

    w0e = jnp.concatenate([W_e_m0, W_e_g0], axis=1)
    b0e = jnp.concatenate([b_e_m0, b_e_g0])[None, :]
    w0n = jnp.concatenate([W_n_m0, W_n_g0], axis=1)
    b0n = jnp.concatenate([b_n_m0, b_n_g0])[None, :]
    weights = [
        w0e.astype(jnp.bfloat16), b0e,
        W_e_m1.astype(jnp.bfloat16), b_e_m1[None, :],
        W_e_g1.astype(jnp.bfloat16), b_e_g1[None, :],
        w0n.astype(jnp.bfloat16), b0n,
        W_n_m1.astype(jnp.bfloat16), b_n_m1[None, :],
        W_n_g1.astype(jnp.bfloat16), b_n_g1[None, :],
        W_ew, W_nw,
    ]

    gathered = {}
    for k in range(min(2, NSLAB)):
        gathered[k] = _sc_step(k, None, node_feat, src, dst, None)

    edge_new = None
    partials = []
    for k in range(NSLAB):
        vi_k, vj_k = gathered.pop(k)
        edge_new, mess_k = _tc_mlp_slab(k, vi_k, vj_k, edge_feat, rbf,
                                        weights, edge_new)
        if k + 2 < NSLAB:
            out = _sc_step(k + 2, k, node_feat, src, dst, mess_k)
            gathered[k + 2] = (out[0], out[1])
            partials.append(out[2])
        else:
            partials.append(_sc_step(None, k, node_feat, src, dst, mess_k))

    node_new = _tc_combine(partials, node_feat)
    return (edge_new, node_new, graph_attr)
